# baseline (speedup 1.0000x reference)
import functools

import jax
import jax.numpy as jnp
from jax import lax
from jax.experimental import pallas as pl
from jax.experimental.pallas import tpu as pltpu

_NSTEPS = 4


def _part(chunk, xrows):
  V = chunk.shape[1]
  col = lax.broadcasted_iota(jnp.int32, chunk.shape, 1)
  picked = jnp.where(col == xrows, chunk, 0.0).sum(axis=1, keepdims=True)
  m = chunk.max(axis=1, keepdims=True)
  s = jnp.exp(chunk - m).sum(axis=1, keepdims=True)
  return jnp.sum(picked - m - jnp.log(s)).reshape(1, 1)


def _tc_body(B, V, R, logits_hbm, auto_ref, xm_ref, xa_ref, out_ref,
             *scratch):
  n = _NSTEPS
  bufs = scratch[:n]
  sems = scratch[n]
  j = pl.program_id(0)

  def stripe_copy(k):
    return pltpu.make_async_copy(
        logits_hbm.at[pl.ds(k * R, R), :], bufs[k], sems.at[k])

  @pl.when(j == 0)
  def _():
    out_ref[...] = jnp.zeros((1, 1), jnp.float32)

  acc = jnp.zeros((1, 1), jnp.float32)
  for k in range(n):
    @pl.when(j == k)
    def _(k=k):
      if k == 0:
        for kk in range(n):
          stripe_copy(kk).start()
      stripe_copy(k).wait()
      part_m = _part(bufs[k][...], xm_ref[...])
      part_a = _part(auto_ref[...], xa_ref[...])
      out_ref[...] += part_m + part_a


def kernel(logits, x):
  B, V = logits.shape
  x = x.astype(jnp.int32)
  xcol = x.reshape(B, 1)

  R = B // (2 * _NSTEPS)
  out = pl.pallas_call(
      functools.partial(_tc_body, B, V, R),
      grid=(_NSTEPS,),
      in_specs=[
          pl.BlockSpec(memory_space=pltpu.MemorySpace.HBM),
          pl.BlockSpec((R, V), lambda j: (j + _NSTEPS, 0)),
          pl.BlockSpec((R, 1), lambda j: (j, 0)),
          pl.BlockSpec((R, 1), lambda j: (j + _NSTEPS, 0)),
      ],
      out_specs=pl.BlockSpec((1, 1), lambda j: (0, 0)),
      out_shape=jax.ShapeDtypeStruct((1, 1), jnp.float32),
      scratch_shapes=(
          [pltpu.VMEM((R, V), jnp.float32) for _ in range(_NSTEPS)]
          + [pltpu.SemaphoreType.DMA((_NSTEPS,))]
      ),
  )(logits, logits, xcol, xcol)
  return out[0, 0]


# speedup vs baseline: 1.0576x; 1.0576x over previous
import functools
import jax
import jax.numpy as jnp
from jax import lax
from jax.experimental import pallas as pl
from jax.experimental.pallas import tpu as pltpu

_STRIPES = (32, 32, 32, 32)

def _tc_body(B, V, stripes, logits_hbm, x_ref, xrep_ref, out_ref, picked,
             psem, *scratch):
  n = len(stripes)
  bufs = scratch[:n]
  sems = scratch[n]
  offs = [sum(stripes[:k]) for k in range(n)]

  def stripe_copy(k):
    return pltpu.make_async_copy(
        logits_hbm.at[pl.ds(offs[k], stripes[k]), :], bufs[k], sems.at[k])

  def pick_copy(r):
    base = pl.multiple_of((x_ref[0, r] // 128) * 128, 128)
    return pltpu.make_async_copy(
        logits_hbm.at[pl.ds((r // 8) * 8, 8), pl.ds(base, 128)],
        picked.at[pl.ds(r * 8, 8), :], psem)

  for k in range(n):
    stripe_copy(k).start()
  for r in range(B):
    pick_copy(r).start()

  total = jnp.zeros((1, 1), jnp.float32)
  for k in range(n):
    stripe_copy(k).wait()
    chunk = bufs[k][...]
    m = chunk.max(axis=1, keepdims=True)
    s = jnp.exp(chunk - m).sum(axis=1, keepdims=True)
    total = total - jnp.sum(m + jnp.log(s)).reshape(1, 1)

  for r in range(B):
    pick_copy(r).wait()
  xr = xrep_ref[...]
  baser = (xr // 128) * 128
  rowi = lax.broadcasted_iota(jnp.int32, (8 * B, 128), 0)
  lane = lax.broadcasted_iota(jnp.int32, (8 * B, 128), 1)
  rowsel = (rowi % 8) == ((rowi // 8) % 8)
  psel = jnp.where(rowsel & (lane == xr - baser), picked[...], 0.0)
  out_ref[...] = total + jnp.sum(psel).reshape(1, 1)

def kernel(logits, x):
  B, V = logits.shape
  x = x.astype(jnp.int32)
  xrep = jnp.repeat(x, 8).reshape(8 * B, 1)
  out = pl.pallas_call(
      functools.partial(_tc_body, B, V, _STRIPES),
      in_specs=[
          pl.BlockSpec(memory_space=pltpu.MemorySpace.HBM),
          pl.BlockSpec(memory_space=pltpu.MemorySpace.SMEM),
          pl.BlockSpec((8 * B, 1), lambda: (0, 0)),
      ],
      out_specs=pl.BlockSpec((1, 1), lambda: (0, 0)),
      out_shape=jax.ShapeDtypeStruct((1, 1), jnp.float32),
      scratch_shapes=(
          [pltpu.VMEM((8 * B, 128), jnp.float32), pltpu.SemaphoreType.DMA]
          + [pltpu.VMEM((r, V), jnp.float32) for r in _STRIPES]
          + [pltpu.SemaphoreType.DMA((len(_STRIPES),))]
      ),
  )(logits, x.reshape(1, B), xrep)
  return out[0, 0]
